# baseline (speedup 1.0000x reference)
import functools

import jax
import jax.numpy as jnp
from jax import lax
from jax.experimental import pallas as pl
from jax.experimental.pallas import tpu as pltpu
from jax.experimental.pallas import tpu_sc as plsc

NUM_ROWS = 32
NUM_COLS = 32
EMBED_DIM = 768
SEQ_LEN = NUM_ROWS * NUM_COLS
LANES = 16

@functools.cache
def _pos_embed_sc():
    mesh = plsc.VectorSubcoreMesh(core_axis_name="c", subcore_axis_name="s")

    @functools.partial(
        pl.kernel,
        mesh=mesh,
        out_type=jax.ShapeDtypeStruct((SEQ_LEN, EMBED_DIM), jnp.float32),
        scratch_types=[
            pltpu.VMEM((1, EMBED_DIM), jnp.float32),
            pltpu.VMEM((NUM_COLS, EMBED_DIM), jnp.float32),
        ],
        compiler_params=pltpu.CompilerParams(use_tc_tiling_on_sc=False),
    )
    def pos_embed_sc(w_row_hbm, w_col_hbm, out_hbm, row_v, out_v):
        wid = lax.axis_index("s") * 2 + lax.axis_index("c")
        pltpu.sync_copy(w_row_hbm.at[pl.ds(wid + 1, 1)], row_v)
        pltpu.sync_copy(w_col_hbm.at[pl.ds(1, NUM_COLS)], out_v)

        def body(c, carry):
            for j in range(EMBED_DIM // LANES):
                sl = pl.ds(j * LANES, LANES)
                out_v[c, sl] = out_v[c, sl] + row_v[0, sl]
            return carry

        lax.fori_loop(0, NUM_COLS, body, 0)
        pltpu.sync_copy(out_v, out_hbm.at[pl.ds(wid * NUM_COLS, NUM_COLS)])

    return pos_embed_sc


def kernel(input, W_row, W_col):
    del input
    out = _pos_embed_sc()(W_row, W_col)
    return out.reshape(1, SEQ_LEN, EMBED_DIM)


# speedup vs baseline: 1.2070x; 1.2070x over previous
import functools

import jax
import jax.numpy as jnp
from jax import lax
from jax.experimental import pallas as pl
from jax.experimental.pallas import tpu as pltpu
from jax.experimental.pallas import tpu_sc as plsc

NUM_ROWS = 32
NUM_COLS = 32
EMBED_DIM = 768
SEQ_LEN = NUM_ROWS * NUM_COLS
LANES = 16

@functools.cache
def _pos_embed_sc():
    mesh = plsc.VectorSubcoreMesh(core_axis_name="c", subcore_axis_name="s")

    @functools.partial(
        pl.kernel,
        mesh=mesh,
        out_type=jax.ShapeDtypeStruct((SEQ_LEN, EMBED_DIM), jnp.float32),
        scratch_types=[
            pltpu.VMEM((1, EMBED_DIM), jnp.float32),
            pltpu.VMEM((NUM_COLS, EMBED_DIM), jnp.float32),
            pltpu.VMEM((NUM_COLS, EMBED_DIM), jnp.float32),
        ],
        compiler_params=pltpu.CompilerParams(use_tc_tiling_on_sc=False),
    )
    def pos_embed_sc(w_row_hbm, w_col_hbm, out_hbm, row_v, col_v, out_v):
        wid = lax.axis_index("s") * 2 + lax.axis_index("c")
        pltpu.sync_copy(w_row_hbm.at[pl.ds(wid + 1, 1)], row_v)
        pltpu.sync_copy(w_col_hbm.at[pl.ds(1, NUM_COLS)], col_v)

        def body(j, carry):
            sl = pl.ds(j * LANES, LANES)
            row_chunk = row_v[0, sl]
            for c in range(NUM_COLS):
                out_v[c, sl] = col_v[c, sl] + row_chunk
            return carry

        lax.fori_loop(0, EMBED_DIM // LANES, body, 0)
        pltpu.sync_copy(out_v, out_hbm.at[pl.ds(wid * NUM_COLS, NUM_COLS)])

    return pos_embed_sc


def kernel(input, W_row, W_col):
    del input
    out = _pos_embed_sc()(W_row, W_col)
    return out.reshape(1, SEQ_LEN, EMBED_DIM)
